# Initial kernel scaffold; baseline (speedup 1.0000x reference)
import jax
import jax.numpy as jnp
from jax.experimental import pallas as pl


def kernel(x, edge_index, edge_weight, edge_index2, edge_weight2, W_ln, b_ln, W1, b1, W2, b2):
    raise NotImplementedError("write your pallas kernel here")


# speedup vs baseline: 2.2865x; 2.2865x over previous
import functools

import jax
import jax.numpy as jnp
from jax import lax
from jax.experimental import pallas as pl
from jax.experimental.pallas import tpu as pltpu
from jax.experimental.pallas import tpu_sc as plsc

HALF = 128
NS = 16
EB = 80
ZR = 200
OW = 1000


@functools.lru_cache(maxsize=None)
def _sc_agg(N, E):
    PT = E // NS
    NB = PT // EB
    NT = N // OW
    NZ = OW // ZR

    mesh = plsc.VectorSubcoreMesh(core_axis_name="c", subcore_axis_name="s")
    out_sds = jax.ShapeDtypeStruct((2, N, HALF), jnp.float32)

    @functools.partial(
        pl.kernel,
        out_type=[out_sds, out_sds],
        mesh=mesh,
        scratch_types=[
            pltpu.VMEM((EB,), jnp.int32),
            pltpu.VMEM((EB,), jnp.int32),
            pltpu.VMEM((EB,), jnp.float32),
            pltpu.VMEM((EB, HALF), jnp.float32),
            pltpu.VMEM((ZR, HALF), jnp.float32),
            pltpu.VMEM_SHARED((N, HALF), jnp.float32),
        ],
    )
    def sc_agg(xs_hbm, src1_hbm, dst1_hbm, w1_hbm, src2_hbm, dst2_hbm, w2_hbm,
               out1_hbm, out2_hbm, srcb, dstb, wvb, rows, zerob, acc):
        c = lax.axis_index("c")
        s = lax.axis_index("s")

        @pl.loop(0, ZR)
        def _zfill(r):
            zrow = zerob.at[r]
            for k in range(HALF // 16):
                zrow[pl.ds(k * 16, 16)] = jnp.zeros((16,), jnp.float32)

        for conv, (src_hbm, dst_hbm, w_hbm, out_hbm) in enumerate([
                (src1_hbm, dst1_hbm, w1_hbm, out1_hbm),
                (src2_hbm, dst2_hbm, w2_hbm, out2_hbm)]):

            @pl.when(s < NT)
            def _zero_stripe():
                @pl.loop(0, NZ)
                def _zero(j):
                    pltpu.sync_copy(zerob, acc.at[pl.ds(s * OW + j * ZR, ZR)])

            plsc.subcore_barrier()

            @pl.loop(0, NB)
            def _block(i):
                base = s * PT + i * EB
                pltpu.sync_copy(src_hbm.at[pl.ds(base, EB)], srcb)
                pltpu.sync_copy(dst_hbm.at[pl.ds(base, EB)], dstb)
                pltpu.sync_copy(w_hbm.at[pl.ds(base, EB)], wvb)
                pltpu.sync_copy(xs_hbm.at[c].at[srcb], rows)

                @pl.loop(0, EB // 16)
                def _scale(g):
                    wv = wvb[pl.ds(g * 16, 16)]
                    for j in range(16):
                        ws = wv[j]
                        rrow = rows.at[g * 16 + j]
                        for k in range(HALF // 16):
                            rrow[pl.ds(k * 16, 16)] = rrow[pl.ds(k * 16, 16)] * ws

                pltpu.sync_copy(rows, acc.at[dstb], add=True)

            plsc.subcore_barrier()

            @pl.when(s < NT)
            def _writeout():
                pltpu.sync_copy(acc.at[pl.ds(s * OW, OW)],
                                out_hbm.at[c].at[pl.ds(s * OW, OW)])

            plsc.subcore_barrier()

    return sc_agg


def _tc_x0_body(x_ref, w_ref, b_ref, o_ref):
    o_ref[...] = jnp.dot(x_ref[...], w_ref[...],
                         preferred_element_type=jnp.float32) + b_ref[...]


def _tc_conv_body(a1l_ref, a1r_ref, a2l_ref, a2r_ref, w1_ref, b1_ref,
                  w2_ref, b2_ref, x1_ref, x2_ref):
    w1t = w1_ref[0:HALF, :]
    w1b = w1_ref[HALF:2 * HALF, :]
    w2t = w2_ref[0:HALF, :]
    w2b = w2_ref[HALF:2 * HALF, :]
    x1_ref[...] = (jnp.dot(a1l_ref[...], w1t, preferred_element_type=jnp.float32)
                   + jnp.dot(a1r_ref[...], w1b, preferred_element_type=jnp.float32)
                   + b1_ref[...])
    x2_ref[...] = (jnp.dot(a2l_ref[...], w2t, preferred_element_type=jnp.float32)
                   + jnp.dot(a2r_ref[...], w2b, preferred_element_type=jnp.float32)
                   + b2_ref[...])


def kernel(x, edge_index, edge_weight, edge_index2, edge_weight2,
           W_ln, b_ln, W1, b1, W2, b2):
    N, D = x.shape
    E = edge_index.shape[1]
    BM = 1000
    grid = (N // BM,)

    xs = jnp.stack([x[:, :HALF], x[:, HALF:]])
    src1 = edge_index[0].astype(jnp.int32)
    dst1 = edge_index[1].astype(jnp.int32)
    src2 = edge_index2[0].astype(jnp.int32)
    dst2 = edge_index2[1].astype(jnp.int32)

    x0 = pl.pallas_call(
        _tc_x0_body,
        grid=grid,
        in_specs=[
            pl.BlockSpec((BM, D), lambda i: (i, 0)),
            pl.BlockSpec((D, D), lambda i: (0, 0)),
            pl.BlockSpec((1, D), lambda i: (0, 0)),
        ],
        out_specs=pl.BlockSpec((BM, D), lambda i: (i, 0)),
        out_shape=jax.ShapeDtypeStruct((N, D), jnp.float32),
    )(x, W_ln, b_ln.reshape(1, D))

    agg1, agg2 = _sc_agg(N, E)(xs, src1, dst1, edge_weight,
                               src2, dst2, edge_weight2)

    half_spec = pl.BlockSpec((BM, HALF), lambda i: (i, 0))
    w_spec = pl.BlockSpec((D, D), lambda i: (0, 0))
    b_spec = pl.BlockSpec((1, D), lambda i: (0, 0))
    out_sds = jax.ShapeDtypeStruct((N, D), jnp.float32)
    x1, x2 = pl.pallas_call(
        _tc_conv_body,
        grid=grid,
        in_specs=[half_spec, half_spec, half_spec, half_spec,
                  w_spec, b_spec, w_spec, b_spec],
        out_specs=[pl.BlockSpec((BM, D), lambda i: (i, 0)),
                   pl.BlockSpec((BM, D), lambda i: (i, 0))],
        out_shape=[out_sds, out_sds],
    )(agg1[0], agg1[1], agg2[0], agg2[1],
      W1, b1.reshape(1, D), W2, b2.reshape(1, D))

    return x0, x1, x2
